# baseline (speedup 1.0000x reference)
import functools

import jax
import jax.numpy as jnp
from jax import lax
from jax.experimental import pallas as pl
from jax.experimental.pallas import tpu as pltpu
from jax.experimental.pallas import tpu_sc as plsc

NC = 2
NS = 16
NW = NC * NS

CHUNK = 512


def _emb_call(B, V, D):
    b_per_w = B // NW
    nchunks = b_per_w // CHUNK
    assert b_per_w % CHUNK == 0 and nchunks % 2 == 0

    mesh = plsc.VectorSubcoreMesh(core_axis_name="c", subcore_axis_name="s")

    @functools.partial(
        pl.kernel,
        mesh=mesh,
        out_type=jax.ShapeDtypeStruct((B, D), jnp.float32),
        scratch_types=[
            pltpu.VMEM((b_per_w,), jnp.int32),
            pltpu.VMEM((2, CHUNK, D), jnp.float32),
            pltpu.SemaphoreType.DMA,
            pltpu.SemaphoreType.DMA,
        ],
        compiler_params=pltpu.CompilerParams(use_tc_tiling_on_sc=False),
    )
    def emb(idx_hbm, w_hbm, out_hbm, idx_v, rows_v, sem0, sem1):
        wid = lax.axis_index("s") * NC + lax.axis_index("c")
        base = wid * b_per_w

        pltpu.sync_copy(idx_hbm.at[pl.ds(base, b_per_w)], idx_v)

        def fire(g, buf, sem):
            pltpu.async_copy(
                w_hbm.at[idx_v.at[pl.ds(g * CHUNK, CHUNK)]],
                rows_v.at[buf],
                sem,
            )

        def drain(buf, sem):
            pltpu.make_async_copy(
                w_hbm.at[pl.ds(0, CHUNK)], rows_v.at[buf], sem
            ).wait()

        fire(0, 0, sem0)
        fire(1, 1, sem1)

        @pl.loop(0, nchunks // 2)
        def _(t):
            g0 = 2 * t
            drain(0, sem0)
            pltpu.sync_copy(
                rows_v.at[0], out_hbm.at[pl.ds(base + g0 * CHUNK, CHUNK)]
            )

            @pl.when(g0 + 2 < nchunks)
            def _():
                fire(g0 + 2, 0, sem0)

            drain(1, sem1)
            pltpu.sync_copy(
                rows_v.at[1], out_hbm.at[pl.ds(base + (g0 + 1) * CHUNK, CHUNK)]
            )

            @pl.when(g0 + 3 < nchunks)
            def _():
                fire(g0 + 3, 1, sem1)

    return emb


@jax.jit
def kernel(x, W):
    B0, S = x.shape
    V, D = W.shape
    B = B0 * S
    idx = x.reshape(B)
    out = _emb_call(B, V, D)(idx, W)
    return out.reshape(B0, S, D)
